# baseline (speedup 1.0000x reference)
import functools

import numpy as np

import jax
import jax.numpy as jnp
from jax import lax
from jax.experimental import pallas as pl
from jax.experimental.pallas import tpu as pltpu
from jax.experimental.pallas import tpu_sc as plsc

N = 4096
E = 8
H = 1024
FF = 4096
T = 256
P = 2 * N + E * T
NT = P // T

NC = 2
NS = 16
NW = NC * NS
APW = 2 * N // NW
TOK = N // NW
GC = 32
NGC = TOK // GC

_GELU_C = float(np.sqrt(2.0 / np.pi))


def _gelu_tanh(x):
    return 0.5 * x * (1.0 + jnp.tanh(_GELU_C * (x + 0.044715 * x * x * x)))


def _router_body(flat_ref, wg_ref, bg_ref,
                 s1_ref, s2_ref, pos_ref, te_ref, lb_ref):
    flat = flat_ref[...]
    wg = wg_ref[...]
    logits = lax.dot_general(flat, wg, (((1,), (1,)), ((), ())),
                             preferred_element_type=jnp.float32)
    logits = logits + bg_ref[...]

    mx = jnp.max(logits, axis=1, keepdims=True)
    ex = jnp.exp(logits - mx)
    probs = ex / jnp.sum(ex, axis=1, keepdims=True)

    colid = lax.broadcasted_iota(jnp.int32, (N, E), 1)
    m1 = jnp.max(probs, axis=1, keepdims=True)
    i1 = jnp.min(jnp.where(probs == m1, colid, E), axis=1, keepdims=True)
    masked = jnp.where(colid == i1, -jnp.inf, probs)
    m2 = jnp.max(masked, axis=1, keepdims=True)
    i2 = jnp.min(jnp.where(masked == m2, colid, E), axis=1, keepdims=True)

    oh1 = (colid == i1).astype(jnp.float32)
    oh2 = (colid == i2).astype(jnp.float32)
    v1 = jnp.sum(probs * oh1, axis=1, keepdims=True)
    v2 = jnp.sum(probs * oh2, axis=1, keepdims=True)
    tot = v1 + v2
    s1_ref[...] = v1 / tot
    s2_ref[...] = v2 / tot

    counts1 = jnp.sum(oh1, axis=0, keepdims=True)
    pm = jnp.sum(probs, axis=0, keepdims=True) / N
    lb_ref[...] = jnp.sum(pm * counts1, axis=1, keepdims=True) * (float(E) / float(N))

    A = 2 * N
    BL = 512
    NB = A // BL
    ohc = jnp.concatenate([oh1, oh2], axis=0)

    seg_r = lax.broadcasted_iota(jnp.int32, (NB, A), 0)
    seg_c = lax.broadcasted_iota(jnp.int32, (NB, A), 1)
    seg = ((seg_c // BL) == seg_r).astype(jnp.float32)
    bs = lax.dot_general(seg, ohc, (((1,), (0,)), ((), ())),
                         preferred_element_type=jnp.float32)
    rb2 = lax.broadcasted_iota(jnp.int32, (NB, NB), 0)
    cb2 = lax.broadcasted_iota(jnp.int32, (NB, NB), 1)
    lb2 = (rb2 > cb2).astype(jnp.float32)
    offs = lax.dot_general(lb2, bs, (((1,), (0,)), ((), ())),
                           preferred_element_type=jnp.float32)

    rb = lax.broadcasted_iota(jnp.int32, (BL, BL), 0)
    cb = lax.broadcasted_iota(jnp.int32, (BL, BL), 1)
    ltri = (rb > cb).astype(jnp.float32)
    pref_blocks = []
    for b in range(NB):
        blk = lax.slice(ohc, (b * BL, 0), ((b + 1) * BL, E))
        pb = lax.dot_general(ltri, blk, (((1,), (0,)), ((), ())),
                             preferred_element_type=jnp.float32)
        pref_blocks.append(pb + lax.slice(offs, (b, 0), (b + 1, E)))
    pref = jnp.concatenate(pref_blocks, axis=0)

    counts = jnp.sum(ohc, axis=0, keepdims=True)
    padded = jnp.ceil(counts / float(T)) * float(T)
    re8 = lax.broadcasted_iota(jnp.int32, (E, E), 0)
    ce8 = lax.broadcasted_iota(jnp.int32, (E, E), 1)
    ustr = (re8 < ce8).astype(jnp.float32)
    pstart = lax.dot_general(padded, ustr, (((1,), (0,)), ((), ())),
                             preferred_element_type=jnp.float32)

    posall = jnp.sum(ohc * (pref + pstart), axis=1, keepdims=True)
    pos_ref[...] = posall.astype(jnp.int32)

    trow = lax.broadcasted_iota(jnp.int32, (NT, E), 0).astype(jnp.float32) * float(T)
    te = jnp.sum((trow >= pstart).astype(jnp.int32), axis=1, keepdims=True) - 1
    te_ref[...] = te


def _router(flat, Wg, bg):
    return pl.pallas_call(
        _router_body,
        out_shape=[
            jax.ShapeDtypeStruct((N, 1), jnp.float32),
            jax.ShapeDtypeStruct((N, 1), jnp.float32),
            jax.ShapeDtypeStruct((2 * N, 1), jnp.int32),
            jax.ShapeDtypeStruct((NT, 1), jnp.int32),
            jax.ShapeDtypeStruct((1, 1), jnp.float32),
        ],
    )(flat, Wg, bg.reshape(1, E))


def _dispatch_body(flat_hbm, pos_hbm, xs_hbm, pos_v, rows_a, rows_b, sem_a, sem_b):
    wid = lax.axis_index("s") * NC + lax.axis_index("c")
    abase = wid * APW
    rowbase = abase % N

    pltpu.sync_copy(pos_hbm.at[pl.ds(abase, APW)], pos_v)

    bufs = (rows_a, rows_b)
    sems = (sem_a, sem_b)
    cps = [None, None]
    ngrp = APW // 16
    for c in range(ngrp):
        if c >= 2:
            cps[c % 2].wait()
        pltpu.sync_copy(flat_hbm.at[pl.ds(rowbase + c * 16, 16)], bufs[c % 2])
        idx = pos_v[pl.ds(c * 16, 16)]
        cps[c % 2] = pltpu.async_copy(bufs[c % 2], xs_hbm.at[idx], sems[c % 2])
    cps[0].wait()
    cps[1].wait()


def _dispatch(flat, posall):
    mesh = plsc.VectorSubcoreMesh(core_axis_name="c", subcore_axis_name="s")
    fn = functools.partial(
        pl.kernel,
        mesh=mesh,
        compiler_params=pltpu.CompilerParams(use_tc_tiling_on_sc=False, needs_layout_passes=False),
        out_type=jax.ShapeDtypeStruct((P, H), jnp.bfloat16),
        scratch_types=[
            pltpu.VMEM((APW,), jnp.int32),
            pltpu.VMEM((16, H), jnp.bfloat16),
            pltpu.VMEM((16, H), jnp.bfloat16),
            pltpu.SemaphoreType.DMA,
            pltpu.SemaphoreType.DMA,
        ],
    )(_dispatch_body)
    return fn(flat, posall)


def _ffn_body(te_ref, x_ref, wg_ref, wu_ref, wd_ref,
              bg_ref, bu_ref, bd_ref, y_ref):
    xb = x_ref[...]
    g = lax.dot_general(xb, wg_ref[0], (((1,), (1,)), ((), ())),
                        preferred_element_type=jnp.float32) + bg_ref[0]
    u = lax.dot_general(xb, wu_ref[0], (((1,), (1,)), ((), ())),
                        preferred_element_type=jnp.float32) + bu_ref[0]
    h = (_gelu_tanh(g) * u).astype(jnp.bfloat16)
    y_ref[...] = lax.dot_general(h, wd_ref[0], (((1,), (1,)), ((), ())),
                                 preferred_element_type=jnp.float32) + bd_ref[0]


def _ffn(te, xs, wgate, wup, wdown, bgate, bup, bdown):
    grid_spec = pltpu.PrefetchScalarGridSpec(
        num_scalar_prefetch=1,
        grid=(NT,),
        in_specs=[
            pl.BlockSpec((T, H), lambda t, te: (t, 0)),
            pl.BlockSpec((1, FF, H), lambda t, te: (te[t], 0, 0)),
            pl.BlockSpec((1, FF, H), lambda t, te: (te[t], 0, 0)),
            pl.BlockSpec((1, H, FF), lambda t, te: (te[t], 0, 0)),
            pl.BlockSpec((1, 1, FF), lambda t, te: (te[t], 0, 0)),
            pl.BlockSpec((1, 1, FF), lambda t, te: (te[t], 0, 0)),
            pl.BlockSpec((1, 1, H), lambda t, te: (te[t], 0, 0)),
        ],
        out_specs=pl.BlockSpec((T, H), lambda t, te: (t, 0)),
    )
    return pl.pallas_call(
        _ffn_body,
        grid_spec=grid_spec,
        out_shape=jax.ShapeDtypeStruct((P, H), jnp.float32),
    )(te, xs, wgate, wup, wdown,
      bgate.reshape(E, 1, FF), bup.reshape(E, 1, FF), bdown.reshape(E, 1, H))


def _combine_body(ys_hbm, pos_hbm, s1_hbm, s2_hbm, out_hbm,
                  p1_v, p2_v, sv_v, y1_v, y2_v, sem1, sem2):
    wid = lax.axis_index("s") * NC + lax.axis_index("c")
    tbase = wid * TOK

    pltpu.sync_copy(pos_hbm.at[pl.ds(tbase, TOK)], p1_v)
    pltpu.sync_copy(pos_hbm.at[pl.ds(N + tbase, TOK)], p2_v)
    pltpu.sync_copy(s1_hbm.at[pl.ds(tbase, TOK)], sv_v.at[0])
    pltpu.sync_copy(s2_hbm.at[pl.ds(tbase, TOK)], sv_v.at[1])

    def grp(c, carry):
        cp1 = pltpu.async_copy(ys_hbm.at[p1_v.at[pl.ds(c * GC, GC)]], y1_v, sem1)
        cp2 = pltpu.async_copy(ys_hbm.at[p2_v.at[pl.ds(c * GC, GC)]], y2_v, sem2)
        cp1.wait()
        cp2.wait()

        def row_add(r, inner):
            lane = jnp.zeros((16,), jnp.int32) + (c * GC + r)
            sp1 = plsc.load_gather(sv_v, [jnp.zeros((16,), jnp.int32), lane])
            sp2 = plsc.load_gather(sv_v, [jnp.zeros((16,), jnp.int32) + 1, lane])
            for v in range(H // 16):
                sl = pl.ds(v * 16, 16)
                y1_v[r, sl] = sp1 * y1_v[r, sl] + sp2 * y2_v[r, sl]
            return inner

        lax.fori_loop(0, GC, row_add, 0)
        pltpu.sync_copy(y1_v, out_hbm.at[pl.ds(tbase + c * GC, GC)])
        return carry

    lax.fori_loop(0, NGC, grp, 0)


def _combine(ys, posall, s1, s2):
    mesh = plsc.VectorSubcoreMesh(core_axis_name="c", subcore_axis_name="s")
    fn = functools.partial(
        pl.kernel,
        mesh=mesh,
        compiler_params=pltpu.CompilerParams(use_tc_tiling_on_sc=False, needs_layout_passes=False),
        out_type=jax.ShapeDtypeStruct((N, H), jnp.float32),
        scratch_types=[
            pltpu.VMEM((TOK,), jnp.int32),
            pltpu.VMEM((TOK,), jnp.int32),
            pltpu.VMEM((2, TOK), jnp.float32),
            pltpu.VMEM((GC, H), jnp.float32),
            pltpu.VMEM((GC, H), jnp.float32),
            pltpu.SemaphoreType.DMA,
            pltpu.SemaphoreType.DMA,
        ],
    )(_combine_body)
    return fn(ys, posall, s1, s2)


def kernel(tensor, Wg, bg, W_gu, b_gu, W_down, b_down):
    B, S, Hd = tensor.shape
    assert B * S == N and Hd == H and Wg.shape == (E, H)
    flat = tensor.reshape(N, H)

    s1, s2, posall, te, lb = _router(flat, Wg, bg)
    posf = posall.reshape(2 * N)

    xs = _dispatch(flat.astype(jnp.bfloat16), posf)

    wgate = W_gu[:, :FF].astype(jnp.bfloat16)
    wup = W_gu[:, FF:].astype(jnp.bfloat16)
    wdown = W_down.astype(jnp.bfloat16)
    bgate = b_gu[:, :FF]
    bup = b_gu[:, FF:]
    ys = _ffn(te.reshape(NT), xs, wgate, wup, wdown, bgate, bup, b_down)

    final = _combine(ys, posf, s1.reshape(N), s2.reshape(N))
    return final.reshape(B, S, Hd), lb.reshape(())


# speedup vs baseline: 1.1094x; 1.1094x over previous
import functools

import numpy as np

import jax
import jax.numpy as jnp
from jax import lax
from jax.experimental import pallas as pl
from jax.experimental.pallas import tpu as pltpu
from jax.experimental.pallas import tpu_sc as plsc

N = 4096
E = 8
H = 1024
FF = 4096
T = 256
P = 2 * N + E * T
NT = P // T

NC = 2
NS = 16
NW = NC * NS
APW = 2 * N // NW
TOK = N // NW
GC = 32
NGC = TOK // GC

_GELU_C = float(np.sqrt(2.0 / np.pi))


def _gelu_tanh(x):
    return 0.5 * x * (1.0 + jnp.tanh(_GELU_C * (x + 0.044715 * x * x * x)))


def _router_body(flat_ref, wg_ref, bg_ref,
                 s1_ref, s2_ref, pos_ref, te_ref, lb_ref):
    flat = flat_ref[...]
    wg = wg_ref[...]
    logits = lax.dot_general(flat, wg, (((1,), (1,)), ((), ())),
                             preferred_element_type=jnp.float32)
    logits = logits + bg_ref[...]

    mx = jnp.max(logits, axis=1, keepdims=True)
    ex = jnp.exp(logits - mx)
    probs = ex / jnp.sum(ex, axis=1, keepdims=True)

    colid = lax.broadcasted_iota(jnp.int32, (N, E), 1)
    m1 = jnp.max(probs, axis=1, keepdims=True)
    i1 = jnp.min(jnp.where(probs == m1, colid, E), axis=1, keepdims=True)
    masked = jnp.where(colid == i1, -jnp.inf, probs)
    m2 = jnp.max(masked, axis=1, keepdims=True)
    i2 = jnp.min(jnp.where(masked == m2, colid, E), axis=1, keepdims=True)

    oh1 = (colid == i1).astype(jnp.float32)
    oh2 = (colid == i2).astype(jnp.float32)
    v1 = jnp.sum(probs * oh1, axis=1, keepdims=True)
    v2 = jnp.sum(probs * oh2, axis=1, keepdims=True)
    tot = v1 + v2
    s1_ref[...] = v1 / tot
    s2_ref[...] = v2 / tot

    counts1 = jnp.sum(oh1, axis=0, keepdims=True)
    pm = jnp.sum(probs, axis=0, keepdims=True) / N
    lb_ref[...] = jnp.sum(pm * counts1, axis=1, keepdims=True) * (float(E) / float(N))

    A = 2 * N
    BL = 512
    NB = A // BL
    ohc = jnp.concatenate([oh1, oh2], axis=0)

    seg_r = lax.broadcasted_iota(jnp.int32, (NB, A), 0)
    seg_c = lax.broadcasted_iota(jnp.int32, (NB, A), 1)
    seg = ((seg_c // BL) == seg_r).astype(jnp.float32)
    bs = lax.dot_general(seg, ohc, (((1,), (0,)), ((), ())),
                         preferred_element_type=jnp.float32)
    rb2 = lax.broadcasted_iota(jnp.int32, (NB, NB), 0)
    cb2 = lax.broadcasted_iota(jnp.int32, (NB, NB), 1)
    lb2 = (rb2 > cb2).astype(jnp.float32)
    offs = lax.dot_general(lb2, bs, (((1,), (0,)), ((), ())),
                           preferred_element_type=jnp.float32)

    rb = lax.broadcasted_iota(jnp.int32, (BL, BL), 0)
    cb = lax.broadcasted_iota(jnp.int32, (BL, BL), 1)
    ltri = (rb > cb).astype(jnp.float32)
    pref_blocks = []
    for b in range(NB):
        blk = lax.slice(ohc, (b * BL, 0), ((b + 1) * BL, E))
        pb = lax.dot_general(ltri, blk, (((1,), (0,)), ((), ())),
                             preferred_element_type=jnp.float32)
        pref_blocks.append(pb + lax.slice(offs, (b, 0), (b + 1, E)))
    pref = jnp.concatenate(pref_blocks, axis=0)

    counts = jnp.sum(ohc, axis=0, keepdims=True)
    padded = jnp.ceil(counts / float(T)) * float(T)
    re8 = lax.broadcasted_iota(jnp.int32, (E, E), 0)
    ce8 = lax.broadcasted_iota(jnp.int32, (E, E), 1)
    ustr = (re8 < ce8).astype(jnp.float32)
    pstart = lax.dot_general(padded, ustr, (((1,), (0,)), ((), ())),
                             preferred_element_type=jnp.float32)

    posall = jnp.sum(ohc * (pref + pstart), axis=1, keepdims=True)
    pos_ref[...] = posall.astype(jnp.int32)

    trow = lax.broadcasted_iota(jnp.int32, (NT, E), 0).astype(jnp.float32) * float(T)
    te = jnp.sum((trow >= pstart).astype(jnp.int32), axis=1, keepdims=True) - 1
    te_ref[...] = te


def _router(flat, Wg, bg):
    return pl.pallas_call(
        _router_body,
        out_shape=[
            jax.ShapeDtypeStruct((N, 1), jnp.float32),
            jax.ShapeDtypeStruct((N, 1), jnp.float32),
            jax.ShapeDtypeStruct((2 * N, 1), jnp.int32),
            jax.ShapeDtypeStruct((NT, 1), jnp.int32),
            jax.ShapeDtypeStruct((1, 1), jnp.float32),
        ],
    )(flat, Wg, bg.reshape(1, E))


def _dispatch_body(flat_hbm, pos_hbm, xs_hbm, pos_v, rows_a, rows_b, sem_a, sem_b):
    wid = lax.axis_index("s") * NC + lax.axis_index("c")
    abase = wid * APW
    rowbase = abase % N

    pltpu.sync_copy(pos_hbm.at[pl.ds(abase, APW)], pos_v)

    bufs = (rows_a, rows_b)
    sems = (sem_a, sem_b)
    cps = [None, None]
    ngrp = APW // 16
    for c in range(ngrp):
        if c >= 2:
            cps[c % 2].wait()
        pltpu.sync_copy(flat_hbm.at[pl.ds(rowbase + c * 16, 16)], bufs[c % 2])
        idx = pos_v[pl.ds(c * 16, 16)]
        cps[c % 2] = pltpu.async_copy(bufs[c % 2], xs_hbm.at[idx], sems[c % 2])
    cps[0].wait()
    cps[1].wait()


def _dispatch(flat, posall):
    mesh = plsc.VectorSubcoreMesh(core_axis_name="c", subcore_axis_name="s")
    fn = functools.partial(
        pl.kernel,
        mesh=mesh,
        compiler_params=pltpu.CompilerParams(use_tc_tiling_on_sc=False, needs_layout_passes=False),
        out_type=jax.ShapeDtypeStruct((P, H), jnp.float32),
        scratch_types=[
            pltpu.VMEM((APW,), jnp.int32),
            pltpu.VMEM((16, H), jnp.float32),
            pltpu.VMEM((16, H), jnp.float32),
            pltpu.SemaphoreType.DMA,
            pltpu.SemaphoreType.DMA,
        ],
    )(_dispatch_body)
    return fn(flat, posall)


def _ffn_body(te_ref, x_ref, wg_ref, wu_ref, wd_ref,
              bg_ref, bu_ref, bd_ref, y_ref):
    xb = x_ref[...].astype(jnp.bfloat16)
    g = lax.dot_general(xb, wg_ref[0], (((1,), (1,)), ((), ())),
                        preferred_element_type=jnp.float32) + bg_ref[0]
    u = lax.dot_general(xb, wu_ref[0], (((1,), (1,)), ((), ())),
                        preferred_element_type=jnp.float32) + bu_ref[0]
    h = (_gelu_tanh(g) * u).astype(jnp.bfloat16)
    y_ref[...] = lax.dot_general(h, wd_ref[0], (((1,), (1,)), ((), ())),
                                 preferred_element_type=jnp.float32) + bd_ref[0]


def _ffn(te, xs, wgate, wup, wdown, bgate, bup, bdown):
    grid_spec = pltpu.PrefetchScalarGridSpec(
        num_scalar_prefetch=1,
        grid=(NT,),
        in_specs=[
            pl.BlockSpec((T, H), lambda t, te: (t, 0)),
            pl.BlockSpec((1, FF, H), lambda t, te: (te[t], 0, 0)),
            pl.BlockSpec((1, FF, H), lambda t, te: (te[t], 0, 0)),
            pl.BlockSpec((1, H, FF), lambda t, te: (te[t], 0, 0)),
            pl.BlockSpec((1, 1, FF), lambda t, te: (te[t], 0, 0)),
            pl.BlockSpec((1, 1, FF), lambda t, te: (te[t], 0, 0)),
            pl.BlockSpec((1, 1, H), lambda t, te: (te[t], 0, 0)),
        ],
        out_specs=pl.BlockSpec((T, H), lambda t, te: (t, 0)),
    )
    return pl.pallas_call(
        _ffn_body,
        grid_spec=grid_spec,
        out_shape=jax.ShapeDtypeStruct((P, H), jnp.float32),
    )(te, xs, wgate, wup, wdown,
      bgate.reshape(E, 1, FF), bup.reshape(E, 1, FF), bdown.reshape(E, 1, H))


def _combine_body(ys_hbm, pos_hbm, s1_hbm, s2_hbm, out_hbm,
                  p1_v, p2_v, sv_v, y1_v, y2_v, sem1, sem2):
    wid = lax.axis_index("s") * NC + lax.axis_index("c")
    tbase = wid * TOK

    pltpu.sync_copy(pos_hbm.at[pl.ds(tbase, TOK)], p1_v)
    pltpu.sync_copy(pos_hbm.at[pl.ds(N + tbase, TOK)], p2_v)
    pltpu.sync_copy(s1_hbm.at[pl.ds(tbase, TOK)], sv_v.at[0])
    pltpu.sync_copy(s2_hbm.at[pl.ds(tbase, TOK)], sv_v.at[1])

    def grp(c, carry):
        cp1 = pltpu.async_copy(ys_hbm.at[p1_v.at[pl.ds(c * GC, GC)]], y1_v, sem1)
        cp2 = pltpu.async_copy(ys_hbm.at[p2_v.at[pl.ds(c * GC, GC)]], y2_v, sem2)
        cp1.wait()
        cp2.wait()

        def row_add(r, inner):
            lane = jnp.zeros((16,), jnp.int32) + (c * GC + r)
            sp1 = plsc.load_gather(sv_v, [jnp.zeros((16,), jnp.int32), lane])
            sp2 = plsc.load_gather(sv_v, [jnp.zeros((16,), jnp.int32) + 1, lane])
            for v in range(H // 16):
                sl = pl.ds(v * 16, 16)
                y1_v[r, sl] = sp1 * y1_v[r, sl] + sp2 * y2_v[r, sl]
            return inner

        lax.fori_loop(0, GC, row_add, 0)
        pltpu.sync_copy(y1_v, out_hbm.at[pl.ds(tbase + c * GC, GC)])
        return carry

    lax.fori_loop(0, NGC, grp, 0)


def _combine(ys, posall, s1, s2):
    mesh = plsc.VectorSubcoreMesh(core_axis_name="c", subcore_axis_name="s")
    fn = functools.partial(
        pl.kernel,
        mesh=mesh,
        compiler_params=pltpu.CompilerParams(use_tc_tiling_on_sc=False, needs_layout_passes=False),
        out_type=jax.ShapeDtypeStruct((N, H), jnp.float32),
        scratch_types=[
            pltpu.VMEM((TOK,), jnp.int32),
            pltpu.VMEM((TOK,), jnp.int32),
            pltpu.VMEM((2, TOK), jnp.float32),
            pltpu.VMEM((GC, H), jnp.float32),
            pltpu.VMEM((GC, H), jnp.float32),
            pltpu.SemaphoreType.DMA,
            pltpu.SemaphoreType.DMA,
        ],
    )(_combine_body)
    return fn(ys, posall, s1, s2)


def kernel(tensor, Wg, bg, W_gu, b_gu, W_down, b_down):
    B, S, Hd = tensor.shape
    assert B * S == N and Hd == H and Wg.shape == (E, H)
    flat = tensor.reshape(N, H)

    s1, s2, posall, te, lb = _router(flat, Wg, bg)
    posf = posall.reshape(2 * N)

    xs = _dispatch(flat, posf)

    wgate = W_gu[:, :FF].astype(jnp.bfloat16)
    wup = W_gu[:, FF:].astype(jnp.bfloat16)
    wdown = W_down.astype(jnp.bfloat16)
    bgate = b_gu[:, :FF]
    bup = b_gu[:, FF:]
    ys = _ffn(te.reshape(NT), xs, wgate, wup, wdown, bgate, bup, b_down)

    final = _combine(ys, posf, s1.reshape(N), s2.reshape(N))
    return final.reshape(B, S, Hd), lb.reshape(())
